# baseline (speedup 1.0000x reference)
import jax
import jax.numpy as jnp
from jax.experimental import pallas as pl

_HIDDEN = 32
_DEPTH = 4
_N_GRAPHS = 4096
_N_NODES = 100000
_N_EDGES = 1600000


def _ident_body(in_ref, out_ref):
    out_ref[...] = in_ref[...]


def kernel(x, edge_x, edge_index, line_edge_index, graph_ids, W_i, W_h, W_o, b_o):
    src = edge_index[0]
    dst = edge_index[1]
    src_x = jnp.take(x, src, axis=0)
    feats = jnp.concatenate([src_x, edge_x], axis=1)
    msg_input = feats @ W_i
    msg = jax.nn.relu(msg_input)
    l_src = line_edge_index[0]
    l_dst = line_edge_index[1]
    for _ in range(_DEPTH - 1):
        accum = jax.ops.segment_sum(jnp.take(msg, l_src, axis=0), l_dst,
                                    num_segments=_N_EDGES)
        msg = jax.nn.relu(msg_input + accum @ W_h)
    m = jax.ops.segment_sum(msg, dst, num_segments=_N_NODES)
    h = jax.nn.relu(jnp.concatenate([x, m], axis=1) @ W_o + b_o)
    sums = jax.ops.segment_sum(h, graph_ids, num_segments=_N_GRAPHS)
    counts = jax.ops.segment_sum(jnp.ones((_N_NODES, 1), dtype=jnp.float32),
                                 graph_ids, num_segments=_N_GRAPHS)
    g_repr = sums / jnp.maximum(counts, 1.0)

    npad = 100096
    xm = jnp.concatenate([x, m], axis=1)
    xmp = jnp.pad(xm, ((0, npad - _N_NODES), (0, 128 - 71)))
    blk = 512
    xmp2 = pl.pallas_call(
        _ident_body,
        grid=(npad // blk,),
        in_specs=[pl.BlockSpec((blk, 128), lambda i: (i, 0))],
        out_specs=pl.BlockSpec((blk, 128), lambda i: (i, 0)),
        out_shape=jax.ShapeDtypeStruct((npad, 128), jnp.float32),
    )(xmp)
    xm2 = xmp2[:_N_NODES, :71]
    h2 = jax.nn.relu(xm2 @ W_o + b_o)
    err = (h2 - h)[: _N_GRAPHS]
    return g_repr + err


# speedup vs baseline: 1.3590x; 1.3590x over previous
import functools

import jax
import jax.numpy as jnp
from jax import lax
from jax.experimental import pallas as pl
from jax.experimental.pallas import tpu as pltpu
from jax.experimental.pallas import tpu_sc as plsc

_H = 32
_W = 128
_DEPTH = 4
_N_GRAPHS = 4096
_N_NODES = 100000
_N_EDGES = 1600000
_N_LINE = 3200000

_NW = 32
_C = 128
_EPAD = 1605632
_EPT = _EPAD // _NW
_NPAD = 100352

_mesh = plsc.VectorSubcoreMesh(core_axis_name="c", subcore_axis_name="s")


def _wid():
    return lax.axis_index("s") * 2 + lax.axis_index("c")


def _gather_rows(vals, idx):
    @functools.partial(
        pl.kernel, mesh=_mesh,
        out_type=jax.ShapeDtypeStruct((_EPAD, _W), jnp.float32),
        scratch_types=[
            pltpu.VMEM((_C,), jnp.int32),
            pltpu.VMEM((_C, _W), jnp.float32),
            pltpu.SemaphoreType.DMA,
        ],
    )
    def k(vals_h, idx_h, out_h, idxbuf, rowbuf, sem):
        base = _wid() * _EPT

        def chunk(g, carry):
            s = base + g * _C
            pltpu.sync_copy(idx_h.at[pl.ds(s, _C)], idxbuf)
            pltpu.async_copy(vals_h.at[idxbuf], rowbuf, sem).wait()
            pltpu.sync_copy(rowbuf, out_h.at[pl.ds(s, _C)])
            return carry

        lax.fori_loop(0, _EPT // _C, chunk, 0)

    return k(vals, idx)


def _make_segsum(dpt, nvt, out_rows, nbp):
    passes = nvt // _NW

    @functools.partial(
        pl.kernel, mesh=_mesh,
        out_type=jax.ShapeDtypeStruct((out_rows * _H,), jnp.float32),
        scratch_types=[
            pltpu.VMEM((nbp,), jnp.int32),
            pltpu.VMEM((_C,), jnp.int32),
            pltpu.VMEM((_C + 16,), jnp.int32),
            pltpu.VMEM((_C, _W), jnp.float32),
            pltpu.VMEM((dpt * _H,), jnp.float32),
            pltpu.SemaphoreType.DMA,
        ],
    )
    def k(vals_h, sidx_h, sdst_h, bounds_h, out_h,
          bbuf, idxbuf, dstbuf, rowbuf, winbuf, sem):
        w = _wid()
        pltpu.sync_copy(bounds_h, bbuf)
        zero16 = jnp.zeros((16,), jnp.float32)

        def do_pass(p, carry):
            vt = p * _NW + w
            d0 = vt * dpt
            bv = bbuf[pl.ds(vt, 16)]
            e0 = bv[0]
            e1 = bv[1]

            def zr(r, c):
                winbuf[pl.ds(r * 16, 16)] = zero16
                return c

            lax.fori_loop(0, dpt * 2, zr, 0)

            c0 = (e0 // _C) * _C

            def chunk(g, c):
                cs = c0 + g * _C
                pltpu.sync_copy(sidx_h.at[pl.ds(cs, _C)], idxbuf)
                pltpu.sync_copy(sdst_h.at[pl.ds(cs, _C)],
                                dstbuf.at[pl.ds(0, _C)])
                pltpu.async_copy(vals_h.at[idxbuf], rowbuf, sem).wait()
                lo = jnp.maximum(e0 - cs, 0)
                hi = jnp.minimum(e1 - cs, _C)

                def edge(j, cc):
                    dv = dstbuf[pl.ds(j, 16)]
                    off = (dv[0] - d0) * _H
                    r0 = rowbuf[j, pl.ds(0, 16)]
                    r1 = rowbuf[j, pl.ds(16, 16)]
                    w0 = winbuf[pl.ds(off, 16)]
                    w1 = winbuf[pl.ds(off + 16, 16)]
                    winbuf[pl.ds(off, 16)] = w0 + r0
                    winbuf[pl.ds(off + 16, 16)] = w1 + r1
                    return cc

                lax.fori_loop(lo, jnp.maximum(lo, hi), edge, 0)
                return c

            nch = (e1 - c0 + _C - 1) // _C
            lax.fori_loop(0, jnp.maximum(nch, 0), chunk, 0)
            pltpu.sync_copy(winbuf, out_h.at[pl.ds(d0 * _H, dpt * _H)])
            return carry

        lax.fori_loop(0, passes, do_pass, 0)

    return k


def _unrolled_mm(x, w, kdim):
    acc = x[:, 0:1] * w[0:1, :]
    for t in range(1, kdim):
        acc = acc + x[:, t:t + 1] * w[t:t + 1, :]
    return acc


def _dot_f32x3(a, b):
    a_hi = a.astype(jnp.bfloat16)
    a_lo = (a - a_hi.astype(jnp.float32)).astype(jnp.bfloat16)
    b_hi = b.astype(jnp.bfloat16)
    b_lo = (b - b_hi.astype(jnp.float32)).astype(jnp.bfloat16)
    dot = lambda u, v: jnp.dot(u, v, preferred_element_type=jnp.float32)
    return dot(a_hi, b_hi) + (dot(a_hi, b_lo) + dot(a_lo, b_hi))


def _pad_w(v):
    return jnp.concatenate(
        [v, jnp.zeros((v.shape[0], _W - _H), jnp.float32)], axis=1)


def _xw_body(x_ref, w_ref, o_ref):
    o_ref[...] = _pad_w(_unrolled_mm(x_ref[...], w_ref[...], 39))


def _ew_body(x_ref, w_ref, o_ref):
    o_ref[...] = _unrolled_mm(x_ref[...], w_ref[...], 11)


def _t1_body(g_ref, e_ref, mi_ref, msg_ref):
    mi = g_ref[:, :_H] + e_ref[...]
    mi_ref[...] = mi
    msg_ref[...] = _pad_w(jnp.maximum(mi, 0.0))


def _t2_body(mi_ref, a_ref, wh_ref, msg_ref):
    r = jnp.maximum(mi_ref[...] + _dot_f32x3(a_ref[...], wh_ref[...]), 0.0)
    msg_ref[...] = _pad_w(r)


def _t3_body(x_ref, m_ref, wot_ref, wob_ref, bo_ref, h_ref):
    h = (_unrolled_mm(x_ref[...], wot_ref[...], 39)
         + _dot_f32x3(m_ref[...], wob_ref[...]) + bo_ref[...])
    h_ref[...] = _pad_w(jnp.maximum(h, 0.0))


def _t4_body(s_ref, r_ref, g_ref):
    g_ref[...] = s_ref[...] * r_ref[...]


def _bspec(blk, width):
    return pl.BlockSpec((blk, width), lambda i: (i, 0))


def _wspec(shape):
    return pl.BlockSpec(shape, lambda i: (0, 0))


def kernel(x, edge_x, edge_index, line_edge_index, graph_ids, W_i, W_h, W_o, b_o):
    f32, i32 = jnp.float32, jnp.int32
    src = edge_index[0]
    dst = edge_index[1]
    l_src = line_edge_index[0]
    l_dst = line_edge_index[1]

    perm_l = jnp.argsort(l_dst)
    sl_src = jnp.take(l_src, perm_l).astype(i32)
    sl_dst = jnp.take(l_dst, perm_l).astype(i32)
    lbounds = jnp.searchsorted(
        sl_dst, jnp.arange(513, dtype=i32) * 3136).astype(i32)
    lbounds = jnp.pad(lbounds, (0, 528 - 513))
    sl_src = jnp.pad(sl_src, (0, 3200128 - _N_LINE))
    sl_dst = jnp.pad(sl_dst, (0, 3200128 - _N_LINE))

    perm_e = jnp.argsort(dst)
    se_dst = jnp.take(dst, perm_e).astype(i32)
    ebounds = jnp.searchsorted(
        se_dst, jnp.arange(33, dtype=i32) * 3136).astype(i32)
    ebounds = jnp.pad(ebounds, (0, 56 - 33))
    se_idx = jnp.pad(perm_e.astype(i32), (0, 1600128 - _N_EDGES))
    se_dst = jnp.pad(se_dst, (0, 1600128 - _N_EDGES))

    gbounds = jnp.searchsorted(
        graph_ids, jnp.arange(33, dtype=i32) * 128).astype(i32)
    gbounds = jnp.pad(gbounds, (0, 56 - 33))
    g_idx = jnp.arange(_NPAD, dtype=i32)
    g_dst = jnp.clip(jnp.pad(graph_ids.astype(i32), (0, _NPAD - _N_NODES)),
                     0, _N_GRAPHS - 1)
    cptr = jnp.searchsorted(graph_ids, jnp.arange(_N_GRAPHS + 1, dtype=i32))
    counts = (cptr[1:] - cptr[:-1]).astype(f32)
    rcp = (1.0 / jnp.maximum(counts, 1.0)).reshape(_N_GRAPHS, 1)

    src_pad = jnp.pad(src.astype(i32), (0, _EPAD - _N_EDGES))

    xp = jnp.pad(x, ((0, _NPAD - _N_NODES), (0, 0)))
    exp_ = jnp.pad(edge_x, ((0, _EPAD - _N_EDGES), (0, 0)))
    xw = pl.pallas_call(
        _xw_body, grid=(_NPAD // 512,),
        in_specs=[_bspec(512, 39), _wspec((39, _H))],
        out_specs=_bspec(512, _W),
        out_shape=jax.ShapeDtypeStruct((_NPAD, _W), f32))(xp, W_i[:39])
    ew = pl.pallas_call(
        _ew_body, grid=(_EPAD // 512,),
        in_specs=[_bspec(512, 11), _wspec((11, _H))],
        out_specs=_bspec(512, _H),
        out_shape=jax.ShapeDtypeStruct((_EPAD, _H), f32))(exp_, W_i[39:])

    gathered = _gather_rows(xw, src_pad)
    msg_input, msg = pl.pallas_call(
        _t1_body, grid=(_EPAD // 512,),
        in_specs=[_bspec(512, _W), _bspec(512, _H)],
        out_specs=[_bspec(512, _H), _bspec(512, _W)],
        out_shape=[jax.ShapeDtypeStruct((_EPAD, _H), f32),
                   jax.ShapeDtypeStruct((_EPAD, _W), f32)])(gathered, ew)

    line_segsum = _make_segsum(3136, 512, _EPAD, 528)
    for _ in range(_DEPTH - 1):
        accum = line_segsum(msg, sl_src, sl_dst, lbounds).reshape(_EPAD, _H)
        msg = pl.pallas_call(
            _t2_body, grid=(_EPAD // 512,),
            in_specs=[_bspec(512, _H), _bspec(512, _H), _wspec((_H, _H))],
            out_specs=_bspec(512, _W),
            out_shape=jax.ShapeDtypeStruct((_EPAD, _W), f32))(
                msg_input, accum, W_h)

    m_segsum = _make_segsum(3136, 32, _NPAD, 56)
    m = m_segsum(msg, se_idx, se_dst, ebounds).reshape(_NPAD, _H)

    h = pl.pallas_call(
        _t3_body, grid=(_NPAD // 512,),
        in_specs=[_bspec(512, 39), _bspec(512, _H), _wspec((39, _H)),
                  _wspec((_H, _H)), _wspec((1, _H))],
        out_specs=_bspec(512, _W),
        out_shape=jax.ShapeDtypeStruct((_NPAD, _W), f32))(
            xp, m, W_o[:39], W_o[39:], b_o.reshape(1, _H))

    g_segsum = _make_segsum(128, 32, _N_GRAPHS, 56)
    sums = g_segsum(h, g_idx, g_dst, gbounds).reshape(_N_GRAPHS, _H)
    g_repr = pl.pallas_call(
        _t4_body, grid=(_N_GRAPHS // 512,),
        in_specs=[_bspec(512, _H), _bspec(512, 1)],
        out_specs=_bspec(512, _H),
        out_shape=jax.ShapeDtypeStruct((_N_GRAPHS, _H), f32))(sums, rcp)
    return g_repr
